# baseline (speedup 1.0000x reference)
import math

import jax
import jax.numpy as jnp
from jax.experimental import pallas as pl
from jax.experimental.pallas import tpu as pltpu

LEN_LAB = 50
LEN_INP = 30
LEN_MED = 40
LEN_ALL = LEN_LAB + LEN_INP + LEN_MED
DIM_LAB = 64
DIM_INP = 32
DIM_MED = 32
D = DIM_LAB + DIM_INP + DIM_MED
D1 = 64
NEG = -1e30
BB = 32


def _mm(x, w):
    return jax.lax.dot_general(x, w, (((1,), (0,)), ((), ())),
                               preferred_element_type=jnp.float32)


def _mmT(x, y):
    return jax.lax.dot_general(x, y, (((0,), (0,)), ((), ())),
                               preferred_element_type=jnp.float32)


def _mmR(x, y):
    return jax.lax.dot_general(x, y, (((1,), (1,)), ((), ())),
                               preferred_element_type=jnp.float32)


def _leaky(x):
    return jnp.where(x > 0, x, 0.01 * x)


def _fused_kernel(tstar_ref, data_ref, decay_ref, mask_ref,
                  Wblk_ref, bcat_ref, nodes0_ref, G0l_ref, G0i_ref, G0m_ref,
                  We0_ref, We1_ref, Wl1_ref, Wi1_ref, Wm1_ref,
                  Wq_ref, Wk_ref, Wv_ref, Wo_ref,
                  Wbeta_ref, bbeta_ref, Wout_ref, bout_ref,
                  out_ref):
    g = pl.program_id(0)
    f32 = jnp.float32
    dat = data_ref[...]
    T = dat.shape[1]
    flat = dat.reshape(BB * T, LEN_ALL)
    lab = flat[:, :LEN_LAB]
    inp_obs = (flat[:, LEN_LAB:LEN_LAB + LEN_INP] != 0.0).astype(f32)
    med = flat[:, LEN_LAB + LEN_INP:]
    a_l = lab * decay_ref[...].reshape(BB * T, LEN_LAB)
    A = jnp.concatenate([a_l, inp_obs, med], axis=1)
    M = jnp.concatenate([mask_ref[...].reshape(BB * T, LEN_LAB),
                         inp_obs, med], axis=1)

    h_e = _mm(M, Wblk_ref[...]) + bcat_ref[...]

    he_t0 = _mm(h_e, We0_ref[...])
    h_e = _leaky(((he_t0 + _mm(A, G0l_ref[...])) + _mm(A, G0i_ref[...]))
                 + _mm(A, G0m_ref[...]))
    nodes0 = nodes0_ref[...]
    nodes = [
        _leaky(nodes0 + _mmT(A[b * T:(b + 1) * T], he_t0[b * T:(b + 1) * T]))
        for b in range(BB)
    ]

    he_t1 = _mm(h_e, We1_ref[...])
    nodes1 = [nodes[b] + _mmT(A[b * T:(b + 1) * T], he_t1[b * T:(b + 1) * T])
              for b in range(BB)]

    G = 8
    NG = BB // G
    ri = jax.lax.broadcasted_iota(jnp.int32, (G, G * T), 0)
    li = jax.lax.broadcasted_iota(jnp.int32, (G, G * T), 1)
    rs = jax.lax.broadcasted_iota(jnp.int32, (G, G * LEN_ALL), 0)
    ls = jax.lax.broadcasted_iota(jnp.int32, (G, G * LEN_ALL), 1)
    off = ls - rs * LEN_ALL
    strip = (off >= 0) & (off < LEN_ALL)

    lane = jax.lax.broadcasted_iota(jnp.int32, (1, LEN_ALL), 1)
    msk_l = (lane < LEN_LAB).astype(f32)
    msk_i = ((lane >= LEN_LAB) & (lane < LEN_LAB + LEN_INP)).astype(f32)
    msk_m = (lane >= LEN_LAB + LEN_INP).astype(f32)

    Nall = jnp.concatenate(nodes1, axis=0)
    K = _mm(Nall, Wk_ref[...])
    V = _mm(Nall, Wv_ref[...])
    sqrt_d = jnp.sqrt(jnp.float32(D))

    he_rows_g = []
    a_rows_g = []
    for gi in range(NG):
        tcol = jnp.concatenate(
            [jnp.full((1, 1), tstar_ref[g * BB + gi * G + r], jnp.int32)
             for r in range(G)], axis=0)
        OH = (li == ri * T + tcol).astype(f32)
        sl = slice(gi * G * T, (gi + 1) * G * T)
        he_rows_g.append(_mm(OH, he_t1[sl]))
        a_rows_g.append(_mm(OH, A[sl]))
    he_rows = jnp.concatenate(he_rows_g, axis=0)
    a_rows = jnp.concatenate(a_rows_g, axis=0)

    Hs = []
    for b in range(BB):
        nb = nodes[b]
        G1 = jnp.concatenate([
            _mm(nb[:LEN_LAB], Wl1_ref[...]),
            _mm(nb[LEN_LAB:LEN_LAB + LEN_INP], Wi1_ref[...]),
            _mm(nb[LEN_LAB + LEN_INP:], Wm1_ref[...]),
        ], axis=0)
        a_row = a_rows[b:b + 1]
        he_row = he_rows[b:b + 1]
        row = ((he_row + _mm(a_row * msk_l, G1))
               + _mm(a_row * msk_i, G1)) + _mm(a_row * msk_m, G1)
        Hs.append(row)

    H = jnp.concatenate(Hs, axis=0)
    Q = _mm(H, Wq_ref[...])
    ctxs = []
    for gi in range(NG):
        nsl = slice(gi * G * LEN_ALL, (gi + 1) * G * LEN_ALL)
        S = _mmR(Q[gi * G:(gi + 1) * G], K[nsl]) / sqrt_d
        S = jnp.where(strip, S, NEG)
        S = S - jnp.max(S, axis=1, keepdims=True)
        E = jnp.where(strip, jnp.exp(S), 0.0)
        attn = E / jnp.sum(E, axis=1, keepdims=True)
        ctxs.append(_mm(attn, V[nsl]))
    C = jnp.concatenate(ctxs, axis=0)

    h_out = _mm(C + H, Wo_ref[...])
    beta = jnp.tanh(_mm(h_out, Wbeta_ref[...]) + bbeta_ref[...])
    logit = _mm(beta, Wout_ref[...]) + bout_ref[...]
    logit = logit - jnp.max(logit, axis=1, keepdims=True)
    el = jnp.exp(logit)
    p = el / jnp.sum(el, axis=1, keepdims=True)
    out_ref[...] = p.reshape(BB, 1, D)


def kernel(data, decay, time, label, lab_mask, length, pid,
           W_lab, b_lab, W_inp, b_inp, W_med, b_med,
           We0, Wl0, Wi0, Wm0, We1, Wl1, Wi1, Wm1,
           Wq, Wk, Wv, Wo, W_beta, b_beta, W_out, b_out):
    B, T, _ = data.shape
    f32 = jnp.float32
    z = jnp.zeros

    Wblk = jnp.concatenate([
        jnp.concatenate([W_lab, z((LEN_LAB, DIM_INP + DIM_MED), f32)], 1),
        jnp.concatenate([z((LEN_INP, DIM_LAB), f32), W_inp,
                         z((LEN_INP, DIM_MED), f32)], 1),
        jnp.concatenate([z((LEN_MED, DIM_LAB + DIM_INP), f32), W_med], 1),
    ], 0)
    bcat = jnp.concatenate([b_lab, b_inp, b_med]).reshape(1, D)

    nb = jnp.concatenate([
        jnp.concatenate([jnp.broadcast_to(b_lab, (LEN_LAB, DIM_LAB)),
                         z((LEN_LAB, DIM_INP + DIM_MED), f32)], 1),
        jnp.concatenate([z((LEN_INP, DIM_LAB), f32),
                         jnp.broadcast_to(b_inp, (LEN_INP, DIM_INP)),
                         z((LEN_INP, DIM_MED), f32)], 1),
        jnp.concatenate([z((LEN_MED, DIM_LAB + DIM_INP), f32),
                         jnp.broadcast_to(b_med, (LEN_MED, DIM_MED))], 1),
    ], 0)
    nodes0 = Wblk + nb
    G0l = jnp.concatenate([nodes0[:LEN_LAB] @ Wl0,
                           z((LEN_INP + LEN_MED, D), f32)], 0)
    G0i = jnp.concatenate([z((LEN_LAB, D), f32),
                           nodes0[LEN_LAB:LEN_LAB + LEN_INP] @ Wi0,
                           z((LEN_MED, D), f32)], 0)
    G0m = jnp.concatenate([z((LEN_LAB + LEN_INP, D), f32),
                           nodes0[LEN_LAB + LEN_INP:] @ Wm0], 0)

    tstar = (jnp.clip(length, 1, T) - 1).astype(jnp.int32)
    Wout_pad = jnp.concatenate([W_out, z((D1, D - 2), f32)], 1)
    bout_pad = jnp.concatenate([b_out, jnp.full((D - 2,), NEG, f32)]
                               ).reshape(1, D)
    bbeta = b_beta.reshape(1, D1)

    full = lambda shape: pl.BlockSpec(shape, lambda i, s: (0,) * len(shape))
    grid_spec = pltpu.PrefetchScalarGridSpec(
        num_scalar_prefetch=1,
        grid=(B // BB,),
        in_specs=[
            pl.BlockSpec((BB, T, LEN_ALL), lambda i, s: (i, 0, 0)),
            pl.BlockSpec((BB, T, LEN_LAB), lambda i, s: (i, 0, 0)),
            pl.BlockSpec((BB, T, LEN_LAB), lambda i, s: (i, 0, 0)),
            full((LEN_ALL, D)), full((1, D)), full((LEN_ALL, D)),
            full((LEN_ALL, D)), full((LEN_ALL, D)), full((LEN_ALL, D)),
            full((D, D)), full((D, D)), full((D, D)), full((D, D)),
            full((D, D)),
            full((D, D)), full((D, D)), full((D, D)), full((D, D)),
            full((D, D1)), full((1, D1)), full((D1, D)), full((1, D)),
        ],
        out_specs=pl.BlockSpec((BB, 1, D), lambda i, s: (i, 0, 0)),
    )
    out = pl.pallas_call(
        _fused_kernel,
        grid_spec=grid_spec,
        out_shape=jax.ShapeDtypeStruct((B, 1, D), f32),
    )(tstar, data, decay, lab_mask, Wblk, bcat, nodes0, G0l, G0i, G0m,
      We0, We1, Wl1, Wi1, Wm1, Wq, Wk, Wv, Wo,
      W_beta, bbeta, Wout_pad, bout_pad)
    return (out[:, 0, :2], label)
